# baseline (speedup 1.0000x reference)
import functools

import jax
import jax.numpy as jnp
from jax import lax
from jax.experimental import pallas as pl
from jax.experimental.pallas import tpu as pltpu
from jax.experimental.pallas import tpu_sc as plsc

D_MODEL = 64
NUM_CORES = 2
NUM_SUBCORES = 16
NW = NUM_CORES * NUM_SUBCORES


def _make_embed(n_rows: int, seq: int):
    rows_per_w = n_rows // NW
    NBUF = 4
    PF = 3
    n_chunks = rows_per_w

    mesh = plsc.VectorSubcoreMesh(core_axis_name="c", subcore_axis_name="s")

    @functools.partial(
        pl.kernel,
        mesh=mesh,
        out_type=jax.ShapeDtypeStruct((n_rows * seq, D_MODEL), jnp.float32),
        compiler_params=pltpu.CompilerParams(use_tc_tiling_on_sc=False),
        scratch_types=[
            pltpu.VMEM((rows_per_w, seq), jnp.int32),
            pltpu.VMEM((NBUF, seq, D_MODEL), jnp.float32),
        ]
        + [pltpu.SemaphoreType.DMA] * (2 * NBUF),
    )
    def embed(tokens_hbm, table_hbm, out_hbm, idx_v, rows_v, *sems):
        gsem = sems[:NBUF]
        osem = sems[NBUF:]
        wid = lax.axis_index("s") * NUM_CORES + lax.axis_index("c")
        base = wid * rows_per_w
        pltpu.sync_copy(tokens_hbm.at[pl.ds(base, rows_per_w)], idx_v)

        def gather(j):
            b = j % NBUF
            pltpu.async_copy(table_hbm.at[idx_v.at[j]], rows_v.at[b], gsem[b])

        def wait_gather(j, b):
            pltpu.make_async_copy(
                table_hbm.at[idx_v.at[j]], rows_v.at[b], gsem[b]
            ).wait()

        def out_start(j, b):
            pltpu.async_copy(
                rows_v.at[b], out_hbm.at[pl.ds((base + j) * seq, seq)], osem[b]
            )

        def wait_out(j, b):
            pltpu.make_async_copy(
                rows_v.at[b], out_hbm.at[pl.ds((base + j) * seq, seq)], osem[b]
            ).wait()

        for j in range(PF):
            gather(j)

        head = NBUF - PF
        for i in range(head):
            gather(i + PF)
            wait_gather(i, i % NBUF)
            out_start(i, i % NBUF)

        n_main = n_chunks - PF - head

        def group(g, carry):
            for b in range(NBUF):
                i = head + g * NBUF + b
                bi = (head + b) % NBUF
                bpf = (head + b + PF) % NBUF
                wait_out(i - head, bpf)
                pltpu.async_copy(
                    table_hbm.at[idx_v.at[i + PF]], rows_v.at[bpf], gsem[bpf]
                )
                wait_gather(i, bi)
                out_start(i, bi)
            return carry

        lax.fori_loop(0, n_main // NBUF, group, 0)

        for i in range(n_chunks - PF, n_chunks):
            wait_gather(i, i % NBUF)
            out_start(i, i % NBUF)

        for j in range(n_chunks - NBUF, n_chunks):
            wait_out(j, j % NBUF)

    return embed


def _make_retile(n_rows: int, seq: int):
    rows_per_w = n_rows // NW
    BCH = 2
    n_chunks = rows_per_w // BCH
    blk = seq * D_MODEL
    lpc = BCH * blk // 128

    mesh = plsc.VectorSubcoreMesh(core_axis_name="c", subcore_axis_name="s")

    @functools.partial(
        pl.kernel,
        mesh=mesh,
        out_type=jax.ShapeDtypeStruct((n_rows, seq, D_MODEL), jnp.float32),
        compiler_params=pltpu.CompilerParams(use_tc_tiling_on_sc=True),
        scratch_types=[
            pltpu.VMEM((2, lpc, 128), jnp.float32),
            pltpu.VMEM((BCH, seq, D_MODEL), jnp.float32),
            pltpu.SemaphoreType.DMA,
            pltpu.SemaphoreType.DMA,
            pltpu.SemaphoreType.DMA,
        ],
    )
    def retile(flat_hbm, out_hbm, bin_, bout, i0, i1, osem):
        isem = (i0, i1)
        wid = lax.axis_index("s") * NUM_CORES + lax.axis_index("c")
        base = wid * rows_per_w

        def in_src(j):
            return flat_hbm.at[pl.ds((base + j * BCH) * (blk // 128), lpc)]

        def in_start(j, b):
            pltpu.async_copy(in_src(j), bin_.at[b], isem[b])

        def wait_in(j, b):
            pltpu.make_async_copy(in_src(j), bin_.at[b], isem[b]).wait()

        def out_dst(j):
            return out_hbm.at[pl.ds(base + j * BCH, BCH)]

        def out_start(j):
            pltpu.async_copy(bout, out_dst(j), osem)

        def wait_out(j):
            pltpu.make_async_copy(bout, out_dst(j), osem).wait()

        def repack(b):
            bo2 = bout.reshape(BCH * seq, D_MODEL)

            def row(r, carry):
                for h in range(2):
                    for v in range(D_MODEL // 16):
                        x = bin_[b, r, pl.ds(h * 64 + v * 16, 16)]
                        bo2[2 * r + h, pl.ds(v * 16, 16)] = x
                return carry

            lax.fori_loop(0, lpc, row, 0)

        in_start(0, 0)
        in_start(1, 1)
        wait_in(0, 0)
        repack(0)
        out_start(0)

        def group(g, carry):
            for j_off, b in ((1, 1), (2, 0)):
                j = 2 * g + j_off
                in_start(j + 1, 1 - b)
                wait_in(j, b)
                wait_out(j - 1)
                repack(b)
                out_start(j)
            return carry

        lax.fori_loop(0, (n_chunks - 2) // 2, group, 0)

        j = n_chunks - 1
        wait_in(j, j % 2)
        wait_out(j - 1)
        repack(j % 2)
        out_start(j)
        wait_out(j)

    return retile


def kernel(tokens, W_E):
    n_rows, seq = tokens.shape
    flat = _make_embed(n_rows, seq)(tokens.astype(jnp.int32), W_E)
    return _make_retile(n_rows, seq)(flat.reshape(-1, 128))


# speedup vs baseline: 1.2318x; 1.2318x over previous
import functools

import jax
import jax.numpy as jnp
from jax import lax
from jax.experimental import pallas as pl
from jax.experimental.pallas import tpu as pltpu
from jax.experimental.pallas import tpu_sc as plsc

D_MODEL = 64
NUM_CORES = 2
NUM_SUBCORES = 16
NW = NUM_CORES * NUM_SUBCORES


def _make_embed(n_rows: int, seq: int):
    rows_per_w = n_rows // NW
    NBUF = 4
    PF = 3
    n_chunks = rows_per_w

    mesh = plsc.VectorSubcoreMesh(core_axis_name="c", subcore_axis_name="s")

    @functools.partial(
        pl.kernel,
        mesh=mesh,
        out_type=jax.ShapeDtypeStruct((n_rows * seq, D_MODEL), jnp.float32),
        compiler_params=pltpu.CompilerParams(use_tc_tiling_on_sc=False),
        scratch_types=[
            pltpu.VMEM((rows_per_w, seq), jnp.int32),
            pltpu.VMEM((NBUF, seq, D_MODEL), jnp.float32),
        ]
        + [pltpu.SemaphoreType.DMA] * (2 * NBUF),
    )
    def embed(tokens_hbm, table_hbm, out_hbm, idx_v, rows_v, *sems):
        gsem = sems[:NBUF]
        osem = sems[NBUF:]
        wid = lax.axis_index("s") * NUM_CORES + lax.axis_index("c")
        base = wid * rows_per_w
        pltpu.sync_copy(tokens_hbm.at[pl.ds(base, rows_per_w)], idx_v)

        def gather(j):
            b = j % NBUF
            pltpu.async_copy(table_hbm.at[idx_v.at[j]], rows_v.at[b], gsem[b])

        def wait_gather(j, b):
            pltpu.make_async_copy(
                table_hbm.at[idx_v.at[j]], rows_v.at[b], gsem[b]
            ).wait()

        def out_start(j, b):
            pltpu.async_copy(
                rows_v.at[b], out_hbm.at[pl.ds((base + j) * seq, seq)], osem[b]
            )

        def wait_out(j, b):
            pltpu.make_async_copy(
                rows_v.at[b], out_hbm.at[pl.ds((base + j) * seq, seq)], osem[b]
            ).wait()

        for j in range(PF):
            gather(j)

        head = NBUF - PF
        for i in range(head):
            gather(i + PF)
            wait_gather(i, i % NBUF)
            out_start(i, i % NBUF)

        n_main = n_chunks - PF - head

        def group(g, carry):
            for b in range(NBUF):
                i = head + g * NBUF + b
                bi = (head + b) % NBUF
                bpf = (head + b + PF) % NBUF
                wait_out(i - head, bpf)
                pltpu.async_copy(
                    table_hbm.at[idx_v.at[i + PF]], rows_v.at[bpf], gsem[bpf]
                )
                wait_gather(i, bi)
                out_start(i, bi)
            return carry

        lax.fori_loop(0, n_main // NBUF, group, 0)

        for i in range(n_chunks - PF, n_chunks):
            wait_gather(i, i % NBUF)
            out_start(i, i % NBUF)

        for j in range(n_chunks - NBUF, n_chunks):
            wait_out(j, j % NBUF)

    return embed


def kernel(tokens, W_E):
    n_rows, seq = tokens.shape
    flat = _make_embed(n_rows, seq)(tokens.astype(jnp.int32), W_E)
    return flat.reshape(n_rows, seq, D_MODEL)
